# baseline (speedup 1.0000x reference)
import functools

import jax
import jax.numpy as jnp
import numpy as np
from jax import lax
from jax.experimental import pallas as pl
from jax.experimental.pallas import tpu as pltpu
from jax.experimental.pallas import tpu_sc as plsc

_NPOSES = 4
_NBLOCKS = 16
_APB = 64
_NATOMS = _NBLOCKS * _APB
_LANES = 16

_CNSTRS = np.array([[0, 0, 2, 0], [0, 0, 2, 0], [0, 0, 2, 0]], dtype=np.int32)
_MAX_N_ATOMS = 4
_CA = np.zeros((3, _MAX_N_ATOMS, 2), dtype=np.int32)
_CA[0, 0] = [0, 0]; _CA[0, 1] = [1, 1]
_CA[1, 0] = [1, 0]; _CA[1, 1] = [2, 1]
_CA[2, 0] = [0, 0]; _CA[2, 1] = [1, 1]
_NC = _CNSTRS.shape[0]


def _lane_const(vals, dtype=np.int32):
    pad = np.zeros(_LANES, dtype=dtype)
    pad[: len(vals)] = vals
    return pad


_POSE = _lane_const(_CNSTRS[:, 1])
_RES_A = _lane_const(_CA[:, 0, 0])
_ATOM_A = _lane_const(_CA[:, 0, 1])
_RES_B = _lane_const(_CA[:, 1, 0])
_ATOM_B = _lane_const(_CA[:, 1, 1])
_ACTIVE = np.zeros(_LANES, dtype=bool)
_ACTIVE[:_NC] = _CNSTRS[:, 0] == 0
_POSES_PRESENT = sorted({int(p) for i, p in enumerate(_CNSTRS[:, 1]) if _ACTIVE[i]})


def _body_sc(coords_hbm, offsets_hbm, out_hbm, coords_v, off_v, scores_v):
    cid = lax.axis_index("c")
    sid = lax.axis_index("s")

    @pl.when(jnp.logical_and(cid == 0, sid == 0))
    def _():
        pltpu.sync_copy(offsets_hbm, off_v)
        pltpu.sync_copy(coords_hbm, coords_v)

        lane = lax.iota(jnp.int32, _LANES)

        def lane_tbl(tbl):
            v = jnp.zeros((_LANES,), jnp.int32)
            for i in range(_NC):
                if tbl[i] != 0:
                    v = jnp.where(lane == i, jnp.int32(int(tbl[i])), v)
            return v

        pose = lane_tbl(_POSE)
        off_a = plsc.load_gather(
            off_v, [pose * _NBLOCKS + lane_tbl(_RES_A)])
        off_b = plsc.load_gather(
            off_v, [pose * _NBLOCKS + lane_tbl(_RES_B)])
        ga = (pose * _NATOMS + off_a + lane_tbl(_ATOM_A)) * 3
        gb = (pose * _NATOMS + off_b + lane_tbl(_ATOM_B)) * 3

        one = jnp.int32(1)
        two = jnp.int32(2)
        dx = plsc.load_gather(coords_v, [ga]) - plsc.load_gather(
            coords_v, [gb])
        dy = plsc.load_gather(coords_v, [ga + one]) - plsc.load_gather(
            coords_v, [gb + one])
        dz = plsc.load_gather(coords_v, [ga + two]) - plsc.load_gather(
            coords_v, [gb + two])

        d2 = jnp.maximum(dx * dx + dy * dy + dz * dz, jnp.float32(1e-30))
        bits = plsc.bitcast(d2, jnp.int32)
        y = plsc.bitcast(jnp.int32(0x5F3759DF) - (bits >> 1), jnp.float32)
        for _ in range(4):
            y = y * (jnp.float32(1.5) - jnp.float32(0.5) * d2 * y * y)
        dist = d2 * y

        t = dist - jnp.float32(4.0)
        harm = t * t

        scores = jnp.zeros((_LANES,), jnp.float32)
        for p in _POSES_PRESENT:
            sel_np = _ACTIVE & (_POSE == p)
            sel = jnp.zeros((_LANES,), jnp.bool_)
            for i in np.nonzero(sel_np)[0]:
                sel = jnp.logical_or(sel, lane == int(i))
            total = jnp.sum(jnp.where(sel, harm, jnp.float32(0.0)))
            onehot = jnp.where(
                lane == p, jnp.float32(1.0), jnp.float32(0.0))
            scores = scores + total * onehot

        scores_v[...] = scores
        pltpu.sync_copy(scores_v, out_hbm)


@functools.cache
def _score_sc():
    return pl.kernel(
        _body_sc,
        mesh=plsc.VectorSubcoreMesh(core_axis_name="c", subcore_axis_name="s"),
        out_type=jax.ShapeDtypeStruct((_LANES,), jnp.float32),
        scratch_types=[
            pltpu.VMEM((_NPOSES * _NATOMS * 3,), jnp.float32),
            pltpu.VMEM((_NPOSES * _NBLOCKS,), jnp.int32),
            pltpu.VMEM((_LANES,), jnp.float32),
        ],
        compiler_params=pltpu.CompilerParams(needs_layout_passes=False),
    )


def kernel(coords, pose_stack_block_coord_offset):
    nposes = pose_stack_block_coord_offset.shape[0]
    flat = _score_sc()(
        coords.reshape(-1),
        pose_stack_block_coord_offset.reshape(-1).astype(jnp.int32),
    )
    return flat[:nposes][None, :]


# speedup vs baseline: 1.0245x; 1.0245x over previous
import functools

import jax
import jax.numpy as jnp
import numpy as np
from jax import lax
from jax.experimental import pallas as pl
from jax.experimental.pallas import tpu as pltpu
from jax.experimental.pallas import tpu_sc as plsc

_NPOSES = 4
_NBLOCKS = 16
_APB = 64
_NATOMS = _NBLOCKS * _APB
_LANES = 16

_CNSTRS = np.array([[0, 0, 2, 0], [0, 0, 2, 0], [0, 0, 2, 0]], dtype=np.int32)
_MAX_N_ATOMS = 4
_CA = np.zeros((3, _MAX_N_ATOMS, 2), dtype=np.int32)
_CA[0, 0] = [0, 0]; _CA[0, 1] = [1, 1]
_CA[1, 0] = [1, 0]; _CA[1, 1] = [2, 1]
_CA[2, 0] = [0, 0]; _CA[2, 1] = [1, 1]
_NC = _CNSTRS.shape[0]


def _lane_const(vals, dtype=np.int32):
    pad = np.zeros(_LANES, dtype=dtype)
    pad[: len(vals)] = vals
    return pad


_POSE = _lane_const(_CNSTRS[:, 1])
_RES_A = _lane_const(_CA[:, 0, 0])
_ATOM_A = _lane_const(_CA[:, 0, 1])
_RES_B = _lane_const(_CA[:, 1, 0])
_ATOM_B = _lane_const(_CA[:, 1, 1])
_ACTIVE = np.zeros(_LANES, dtype=bool)
_ACTIVE[:_NC] = _CNSTRS[:, 0] == 0
_POSES_PRESENT = sorted({int(p) for i, p in enumerate(_CNSTRS[:, 1]) if _ACTIVE[i]})
_POSE_SET = sorted({int(p) for p in _POSE})
_POSE_DENSE = np.array([_POSE_SET.index(int(p)) for p in _POSE], dtype=np.int32)


def _body_sc(coords_hbm, offsets_hbm, out_hbm, coords_v, off_v, scores_v,
             sem_c, sem_o):
    cid = lax.axis_index("c")
    sid = lax.axis_index("s")

    @pl.when(jnp.logical_and(cid == 0, sid == 0))
    def _():
        copy_o = pltpu.make_async_copy(offsets_hbm, off_v, sem_o)
        copy_o.start()
        copies_c = []
        for k, p in enumerate(_POSE_SET):
            c = pltpu.make_async_copy(coords_hbm.at[p], coords_v.at[k], sem_c)
            c.start()
            copies_c.append(c)
        copy_o.wait()
        for c in copies_c:
            c.wait()

        lane = lax.iota(jnp.int32, _LANES)

        def lane_tbl(tbl):
            v = jnp.zeros((_LANES,), jnp.int32)
            for i in range(_NC):
                if tbl[i] != 0:
                    v = jnp.where(lane == i, jnp.int32(int(tbl[i])), v)
            return v

        pose = lane_tbl(_POSE)
        posek = lane_tbl(_POSE_DENSE)
        off_a = plsc.load_gather(off_v, [pose, lane_tbl(_RES_A)])
        off_b = plsc.load_gather(off_v, [pose, lane_tbl(_RES_B)])
        ga = off_a + lane_tbl(_ATOM_A)
        gb = off_b + lane_tbl(_ATOM_B)

        comp0 = jnp.zeros((_LANES,), jnp.int32)
        comp1 = comp0 + jnp.int32(1)
        comp2 = comp0 + jnp.int32(2)
        dx = plsc.load_gather(coords_v, [posek, ga, comp0]) - plsc.load_gather(
            coords_v, [posek, gb, comp0])
        dy = plsc.load_gather(coords_v, [posek, ga, comp1]) - plsc.load_gather(
            coords_v, [posek, gb, comp1])
        dz = plsc.load_gather(coords_v, [posek, ga, comp2]) - plsc.load_gather(
            coords_v, [posek, gb, comp2])

        d2 = jnp.maximum(dx * dx + dy * dy + dz * dz, jnp.float32(1e-30))
        bits = plsc.bitcast(d2, jnp.int32)
        y = plsc.bitcast(jnp.int32(0x5F3759DF) - (bits >> 1), jnp.float32)
        for _ in range(4):
            y = y * (jnp.float32(1.5) - jnp.float32(0.5) * d2 * y * y)
        dist = d2 * y

        t = dist - jnp.float32(4.0)
        harm = t * t

        scores = jnp.zeros((_LANES,), jnp.float32)
        for p in _POSES_PRESENT:
            sel_np = _ACTIVE & (_POSE == p)
            sel = jnp.zeros((_LANES,), jnp.bool_)
            for i in np.nonzero(sel_np)[0]:
                sel = jnp.logical_or(sel, lane == int(i))
            total = jnp.sum(jnp.where(sel, harm, jnp.float32(0.0)))
            onehot = jnp.where(
                lane == p, jnp.float32(1.0), jnp.float32(0.0))
            scores = scores + total * onehot

        scores_v[...] = scores
        pltpu.sync_copy(scores_v.at[pl.ds(0, _NPOSES)], out_hbm.at[0])


@functools.cache
def _score_sc():
    return pl.kernel(
        _body_sc,
        mesh=plsc.VectorSubcoreMesh(core_axis_name="c", subcore_axis_name="s"),
        out_type=jax.ShapeDtypeStruct((1, _NPOSES), jnp.float32),
        scratch_types=[
            pltpu.VMEM((len(_POSE_SET), _NATOMS, 3), jnp.float32),
            pltpu.VMEM((_NPOSES, _NBLOCKS), jnp.int32),
            pltpu.VMEM((_LANES,), jnp.float32),
            pltpu.SemaphoreType.DMA,
            pltpu.SemaphoreType.DMA,
        ],
        compiler_params=pltpu.CompilerParams(
            needs_layout_passes=False, use_tc_tiling_on_sc=False),
    )


def kernel(coords, pose_stack_block_coord_offset):
    return _score_sc()(coords, pose_stack_block_coord_offset)


# speedup vs baseline: 1.0914x; 1.0653x over previous
import functools

import jax
import jax.numpy as jnp
import numpy as np
from jax import lax
from jax.experimental import pallas as pl
from jax.experimental.pallas import tpu as pltpu
from jax.experimental.pallas import tpu_sc as plsc

_NPOSES = 4
_NBLOCKS = 16
_APB = 64
_NATOMS = _NBLOCKS * _APB
_LANES = 16

_CNSTRS = np.array([[0, 0, 2, 0], [0, 0, 2, 0], [0, 0, 2, 0]], dtype=np.int32)
_MAX_N_ATOMS = 4
_CA = np.zeros((3, _MAX_N_ATOMS, 2), dtype=np.int32)
_CA[0, 0] = [0, 0]; _CA[0, 1] = [1, 1]
_CA[1, 0] = [1, 0]; _CA[1, 1] = [2, 1]
_CA[2, 0] = [0, 0]; _CA[2, 1] = [1, 1]
_NC = _CNSTRS.shape[0]


def _lane_const(vals, dtype=np.int32):
    pad = np.zeros(_LANES, dtype=dtype)
    pad[: len(vals)] = vals
    return pad


_POSE = _lane_const(_CNSTRS[:, 1])
_RES_A = _lane_const(_CA[:, 0, 0])
_ATOM_A = _lane_const(_CA[:, 0, 1])
_RES_B = _lane_const(_CA[:, 1, 0])
_ATOM_B = _lane_const(_CA[:, 1, 1])
_ACTIVE = np.zeros(_LANES, dtype=bool)
_ACTIVE[:_NC] = _CNSTRS[:, 0] == 0
_POSES_PRESENT = sorted({int(p) for i, p in enumerate(_CNSTRS[:, 1]) if _ACTIVE[i]})
_POSE_SET = sorted({int(p) for p in _POSE})
_POSE_DENSE = np.array([_POSE_SET.index(int(p)) for p in _POSE], dtype=np.int32)


def _body_sc(coords_hbm, offsets_hbm, out_hbm, coords_v, off_v, scores_v,
             sem_c, sem_o):
    cid = lax.axis_index("c")
    sid = lax.axis_index("s")

    @pl.when(jnp.logical_and(cid == 0, sid == 0))
    def _():
        copy_o = pltpu.make_async_copy(offsets_hbm, off_v, sem_o)
        copy_o.start()
        copies_c = []
        for k, p in enumerate(_POSE_SET):
            c = pltpu.make_async_copy(coords_hbm.at[p], coords_v.at[k], sem_c)
            c.start()
            copies_c.append(c)
        copy_o.wait()
        for c in copies_c:
            c.wait()

        lane = lax.iota(jnp.int32, _LANES)

        def lane_tbl(tbl):
            v = jnp.zeros((_LANES,), jnp.int32)
            for i in range(_NC):
                if tbl[i] != 0:
                    v = jnp.where(lane == i, jnp.int32(int(tbl[i])), v)
            return v

        pose = lane_tbl(_POSE)
        posek = lane_tbl(_POSE_DENSE)
        off_a = plsc.load_gather(off_v, [pose, lane_tbl(_RES_A)])
        off_b = plsc.load_gather(off_v, [pose, lane_tbl(_RES_B)])
        ga = off_a + lane_tbl(_ATOM_A)
        gb = off_b + lane_tbl(_ATOM_B)

        comp0 = jnp.zeros((_LANES,), jnp.int32)
        comp1 = comp0 + jnp.int32(1)
        comp2 = comp0 + jnp.int32(2)
        dx = plsc.load_gather(coords_v, [posek, ga, comp0]) - plsc.load_gather(
            coords_v, [posek, gb, comp0])
        dy = plsc.load_gather(coords_v, [posek, ga, comp1]) - plsc.load_gather(
            coords_v, [posek, gb, comp1])
        dz = plsc.load_gather(coords_v, [posek, ga, comp2]) - plsc.load_gather(
            coords_v, [posek, gb, comp2])

        d2 = jnp.maximum(dx * dx + dy * dy + dz * dz, jnp.float32(1e-30))
        bits = plsc.bitcast(d2, jnp.int32)
        y = plsc.bitcast(jnp.int32(0x5F3759DF) - (bits >> 1), jnp.float32)
        for _ in range(3):
            y = y * (jnp.float32(1.5) - jnp.float32(0.5) * d2 * y * y)
        dist = d2 * y

        t = dist - jnp.float32(4.0)
        harm = t * t

        scores = jnp.zeros((_LANES,), jnp.float32)
        for p in _POSES_PRESENT:
            sel_np = _ACTIVE & (_POSE == p)
            sel = jnp.zeros((_LANES,), jnp.bool_)
            for i in np.nonzero(sel_np)[0]:
                sel = jnp.logical_or(sel, lane == int(i))
            total = jnp.sum(jnp.where(sel, harm, jnp.float32(0.0)))
            onehot = jnp.where(
                lane == p, jnp.float32(1.0), jnp.float32(0.0))
            scores = scores + total * onehot

        scores_v[...] = scores
        pltpu.sync_copy(scores_v.at[pl.ds(0, _NPOSES)], out_hbm.at[0])


@functools.cache
def _score_sc():
    return pl.kernel(
        _body_sc,
        mesh=plsc.VectorSubcoreMesh(
            core_axis_name="c", subcore_axis_name="s", num_cores=1),
        out_type=jax.ShapeDtypeStruct((1, _NPOSES), jnp.float32),
        scratch_types=[
            pltpu.VMEM((len(_POSE_SET), _NATOMS, 3), jnp.float32),
            pltpu.VMEM((_NPOSES, _NBLOCKS), jnp.int32),
            pltpu.VMEM((_LANES,), jnp.float32),
            pltpu.SemaphoreType.DMA,
            pltpu.SemaphoreType.DMA,
        ],
        compiler_params=pltpu.CompilerParams(
            needs_layout_passes=False, use_tc_tiling_on_sc=False),
    )


def kernel(coords, pose_stack_block_coord_offset):
    return _score_sc()(coords, pose_stack_block_coord_offset)


# speedup vs baseline: 1.0971x; 1.0052x over previous
import functools

import jax
import jax.numpy as jnp
import numpy as np
from jax import lax
from jax.experimental import pallas as pl
from jax.experimental.pallas import tpu as pltpu
from jax.experimental.pallas import tpu_sc as plsc

_NPOSES = 4
_NBLOCKS = 16
_APB = 64
_NATOMS = _NBLOCKS * _APB
_LANES = 16

_CNSTRS = np.array([[0, 0, 2, 0], [0, 0, 2, 0], [0, 0, 2, 0]], dtype=np.int32)
_MAX_N_ATOMS = 4
_CA = np.zeros((3, _MAX_N_ATOMS, 2), dtype=np.int32)
_CA[0, 0] = [0, 0]; _CA[0, 1] = [1, 1]
_CA[1, 0] = [1, 0]; _CA[1, 1] = [2, 1]
_CA[2, 0] = [0, 0]; _CA[2, 1] = [1, 1]
_NC = _CNSTRS.shape[0]


def _lane_const(vals, dtype=np.int32):
    pad = np.zeros(_LANES, dtype=dtype)
    pad[: len(vals)] = vals
    return pad


_POSE = _lane_const(_CNSTRS[:, 1])
_RES_A = _lane_const(_CA[:, 0, 0])
_ATOM_A = _lane_const(_CA[:, 0, 1])
_RES_B = _lane_const(_CA[:, 1, 0])
_ATOM_B = _lane_const(_CA[:, 1, 1])
_ACTIVE = np.zeros(_LANES, dtype=bool)
_ACTIVE[:_NC] = _CNSTRS[:, 0] == 0
_POSES_PRESENT = sorted({int(p) for i, p in enumerate(_CNSTRS[:, 1]) if _ACTIVE[i]})
_POSE_SET = sorted({int(p) for p in _POSE})
_POSE_DENSE = np.array([_POSE_SET.index(int(p)) for p in _POSE], dtype=np.int32)


def _body_sc(coords_hbm, offsets_hbm, out_hbm, coords_v, off_v, scores_v,
             sem_c, sem_o):
    cid = lax.axis_index("c")
    sid = lax.axis_index("s")

    @pl.when(jnp.logical_and(cid == 0, sid == 0))
    def _():
        copy_o = pltpu.make_async_copy(offsets_hbm, off_v, sem_o)
        copy_o.start()
        copies_c = []
        for k, p in enumerate(_POSE_SET):
            c = pltpu.make_async_copy(coords_hbm.at[p], coords_v.at[k], sem_c)
            c.start()
            copies_c.append(c)
        copy_o.wait()
        for c in copies_c:
            c.wait()

        lane = lax.iota(jnp.int32, _LANES)

        def lane_tbl(tbl):
            v = jnp.zeros((_LANES,), jnp.int32)
            for i in range(_NC):
                if tbl[i] != 0:
                    v = jnp.where(lane == i, jnp.int32(int(tbl[i])), v)
            return v

        pose = lane_tbl(_POSE)
        posek = lane_tbl(_POSE_DENSE)
        off_a = plsc.load_gather(off_v, [pose, lane_tbl(_RES_A)])
        off_b = plsc.load_gather(off_v, [pose, lane_tbl(_RES_B)])
        ga = off_a + lane_tbl(_ATOM_A)
        gb = off_b + lane_tbl(_ATOM_B)

        comp0 = jnp.zeros((_LANES,), jnp.int32)
        comp1 = comp0 + jnp.int32(1)
        comp2 = comp0 + jnp.int32(2)
        dx = plsc.load_gather(coords_v, [posek, ga, comp0]) - plsc.load_gather(
            coords_v, [posek, gb, comp0])
        dy = plsc.load_gather(coords_v, [posek, ga, comp1]) - plsc.load_gather(
            coords_v, [posek, gb, comp1])
        dz = plsc.load_gather(coords_v, [posek, ga, comp2]) - plsc.load_gather(
            coords_v, [posek, gb, comp2])

        d2 = jnp.maximum(dx * dx + dy * dy + dz * dz, jnp.float32(1e-30))
        bits = plsc.bitcast(d2, jnp.int32)
        y = plsc.bitcast(jnp.int32(0x5F3759DF) - (bits >> 1), jnp.float32)
        for _ in range(3):
            y = y * (jnp.float32(1.5) - jnp.float32(0.5) * d2 * y * y)
        dist = d2 * y

        t = dist - jnp.float32(4.0)
        harm = t * t

        scores = jnp.zeros((_LANES,), jnp.float32)
        for p in _POSES_PRESENT:
            sel_np = _ACTIVE & (_POSE == p)
            sel = jnp.zeros((_LANES,), jnp.bool_)
            for i in np.nonzero(sel_np)[0]:
                sel = jnp.logical_or(sel, lane == int(i))
            total = jnp.sum(jnp.where(sel, harm, jnp.float32(0.0)))
            onehot = jnp.where(
                lane == p, jnp.float32(1.0), jnp.float32(0.0))
            scores = scores + total * onehot

        scores_v[...] = scores
        pltpu.sync_copy(scores_v.at[pl.ds(0, _NPOSES)], out_hbm.at[0])


@functools.cache
def _score_sc():
    return pl.kernel(
        _body_sc,
        mesh=plsc.VectorSubcoreMesh(
            core_axis_name="c", subcore_axis_name="s", num_cores=1),
        out_type=jax.ShapeDtypeStruct((1, _NPOSES), jnp.float32),
        scratch_types=[
            pltpu.VMEM((len(_POSE_SET), _NATOMS, 3), jnp.float32),
            pltpu.VMEM((_NPOSES, _NBLOCKS), jnp.int32),
            pltpu.VMEM((_LANES,), jnp.float32),
            pltpu.SemaphoreType.DMA,
            pltpu.SemaphoreType.DMA,
        ],
        compiler_params=pltpu.CompilerParams(
            needs_layout_passes=False,
            use_tc_tiling_on_sc=False,
            disable_bounds_checks=True,
            disable_semaphore_checks=True,
            skip_device_barrier=True,
        ),
    )


def kernel(coords, pose_stack_block_coord_offset):
    return _score_sc()(coords, pose_stack_block_coord_offset)
